# baseline (speedup 1.0000x reference)
import functools

import jax
import jax.numpy as jnp
from jax import lax
from jax.experimental import pallas as pl
from jax.experimental.pallas import tpu as pltpu
from jax.experimental.pallas import tpu_sc as plsc

BATCH = 1024
SEQ = 128
VOCAB = 100000
D = 128

NC = 2
NS = 16
NW = NC * NS
TOK = BATCH * SEQ
TPW = TOK // NW
G = 128
KCH = TPW // G


def _sc_body(ids_hbm, maskf_hbm, table_hbm, out_hbm, idx_v, maskf_v, rows_v, gsem):
    c = lax.axis_index("c")
    s = lax.axis_index("s")
    wid = s * NC + c
    base = wid * TPW

    pltpu.sync_copy(ids_hbm.at[wid], idx_v)
    pltpu.sync_copy(maskf_hbm.at[wid], maskf_v)

    def group(g, carry):
        pltpu.async_copy(table_hbm.at[idx_v.at[g]], rows_v, gsem).wait()

        def tok(t, carry2):
            midx = jnp.full((16,), 0, jnp.int32) + (g * G + t)
            m = plsc.load_gather(maskf_v, [midx])
            for j in range(D // 16):
                sl = pl.ds(j * 16, 16)
                rows_v[t, sl] = rows_v[t, sl] * m
            return carry2

        lax.fori_loop(0, G, tok, 0)

        pltpu.sync_copy(rows_v, out_hbm.at[pl.ds(base + g * G, G)])
        return carry

    lax.fori_loop(0, KCH, group, 0)


@jax.jit
def _sc_call(ids, maskf, table):
    mesh = plsc.VectorSubcoreMesh(core_axis_name="c", subcore_axis_name="s")
    kfn = functools.partial(
        pl.kernel,
        mesh=mesh,
        out_type=jax.ShapeDtypeStruct((TOK, D), jnp.float32),
        scratch_types=[
            pltpu.VMEM((KCH, G), jnp.int32),
            pltpu.VMEM((TPW,), jnp.float32),
            pltpu.VMEM((G, D), jnp.float32),
            pltpu.SemaphoreType.DMA,
        ],
        compiler_params=pltpu.CompilerParams(needs_layout_passes=False),
    )(_sc_body)
    return kfn(ids, maskf, table)


def kernel(input_ids, attention_mask, embedding_table):
    ids = input_ids.reshape(NW, KCH, G)
    maskf = attention_mask.reshape(NW, TPW).astype(jnp.float32)
    out = _sc_call(ids, maskf, embedding_table)
    return out.reshape(BATCH, SEQ, D)


# speedup vs baseline: 1.4930x; 1.4930x over previous
import functools

import jax
import jax.numpy as jnp
from jax import lax
from jax.experimental import pallas as pl
from jax.experimental.pallas import tpu as pltpu
from jax.experimental.pallas import tpu_sc as plsc

BATCH = 1024
SEQ = 128
VOCAB = 100000
D = 128

NC = 2
NS = 16
NW = NC * NS
TOK = BATCH * SEQ
TPW = TOK // NW
G = 128
KCH = TPW // G


def _sc_body(ids_hbm, maskf_hbm, table_hbm, out_hbm, idx_v, maskf_v,
             rows0, rows1, gs0, gs1, os0, os1):
    c = lax.axis_index("c")
    s = lax.axis_index("s")
    wid = s * NC + c
    base = wid * TPW

    pltpu.sync_copy(ids_hbm.at[wid], idx_v)
    pltpu.sync_copy(maskf_hbm.at[wid], maskf_v)

    rows = (rows0, rows1)
    gs = (gs0, gs1)
    os = (os0, os1)
    ghandle = [None, None]
    ohandle = [None, None]

    def multiply(rv, g):
        def tok(t, carry2):
            midx = jnp.full((16,), 0, jnp.int32) + (g * G + t)
            m = plsc.load_gather(maskf_v, [midx])
            for j in range(D // 16):
                sl = pl.ds(j * 16, 16)
                rv[t, sl] = rv[t, sl] * m
            return carry2

        lax.fori_loop(0, G, tok, 0)

    ghandle[0] = pltpu.async_copy(table_hbm.at[idx_v.at[0]], rows[0], gs[0])
    for g in range(KCH):
        b = g % 2
        if g + 1 < KCH:
            b2 = (g + 1) % 2
            if ohandle[b2] is not None:
                ohandle[b2].wait()
                ohandle[b2] = None
            ghandle[b2] = pltpu.async_copy(
                table_hbm.at[idx_v.at[g + 1]], rows[b2], gs[b2])
        ghandle[b].wait()
        multiply(rows[b], g)
        ohandle[b] = pltpu.async_copy(
            rows[b], out_hbm.at[pl.ds(base + g * G, G)], os[b])
    for b in range(2):
        if ohandle[b] is not None:
            ohandle[b].wait()


@jax.jit
def _sc_call(ids, maskf, table):
    mesh = plsc.VectorSubcoreMesh(core_axis_name="c", subcore_axis_name="s")
    kfn = functools.partial(
        pl.kernel,
        mesh=mesh,
        out_type=jax.ShapeDtypeStruct((TOK, D), jnp.float32),
        scratch_types=[
            pltpu.VMEM((KCH, G), jnp.int32),
            pltpu.VMEM((TPW,), jnp.float32),
            pltpu.VMEM((G, D), jnp.float32),
            pltpu.VMEM((G, D), jnp.float32),
            pltpu.SemaphoreType.DMA,
            pltpu.SemaphoreType.DMA,
            pltpu.SemaphoreType.DMA,
            pltpu.SemaphoreType.DMA,
        ],
        compiler_params=pltpu.CompilerParams(needs_layout_passes=False),
    )(_sc_body)
    return kfn(ids, maskf, table)


def kernel(input_ids, attention_mask, embedding_table):
    ids = input_ids.reshape(NW, KCH, G)
    maskf = attention_mask.reshape(NW, TPW).astype(jnp.float32)
    out = _sc_call(ids, maskf, embedding_table)
    return out.reshape(BATCH, SEQ, D)
